# baseline (speedup 1.0000x reference)
import functools

import jax
import jax.numpy as jnp
from jax import lax
from jax.experimental import pallas as pl
from jax.experimental.pallas import tpu as pltpu
from jax.experimental.pallas import tpu_sc as plsc

N = 16384
V = 100000
LATENT = 16
NUM_METABOLITES = 128

NUM_CORES = 2
NUM_SUBCORES = 16
NW = NUM_CORES * NUM_SUBCORES
BPW = N // NW
NCH = BPW // 128

_SC_MESH = plsc.VectorSubcoreMesh(
    core_axis_name="c", subcore_axis_name="s",
    num_cores=NUM_CORES, num_subcores=NUM_SUBCORES,
)


def _build_offsets(idx_v, idxs_v, nk):
    def off_body(r):
        j = r // nk
        k = r - j * nk
        for c in range(8):
            sl = pl.ds(c * 16, 16)
            idxs_v[r, sl] = idx_v[j, sl] + k * V
    plsc.parallel_loop(0, nk * NCH, 1, unroll=2)(off_body)

HALF = LATENT // 2


def _sc_gather_mu(x_hbm, tmu_hbm, mu_out, idx_v, idxs_v, mu_v, sem):
    wid = lax.axis_index("s") * NUM_CORES + lax.axis_index("c")
    pltpu.sync_copy(x_hbm.at[pl.ds(wid * NCH, NCH)], idx_v)
    _build_offsets(idx_v, idxs_v, HALF)

    def fire(r, carry):
        pltpu.async_copy(tmu_hbm.at[idxs_v.at[r]], mu_v.at[r], sem)
        return carry
    lax.fori_loop(0, HALF * NCH, fire, 0, unroll=4)
    pltpu.make_async_copy(mu_out.at[wid], mu_v, sem).wait()
    pltpu.sync_copy(mu_v, mu_out.at[wid])


def _sc_gather_lv(x_hbm, tlv_hbm, bias_mu_hbm, bias_lv_hbm,
                  lv_out, bmu_out, blv_out,
                  idx_v, idxs_v, lv_v, bmu_v, blv_v, sem):
    wid = lax.axis_index("s") * NUM_CORES + lax.axis_index("c")
    pltpu.sync_copy(x_hbm.at[pl.ds(wid * NCH, NCH)], idx_v)
    _build_offsets(idx_v, idxs_v, LATENT)

    def fire(r, carry):
        pltpu.async_copy(tlv_hbm.at[idxs_v.at[r]], lv_v.at[r], sem)
        return carry
    lax.fori_loop(0, LATENT * NCH, fire, 0, unroll=4)
    for j in range(NCH):
        pltpu.async_copy(bias_mu_hbm.at[idx_v.at[j]], bmu_v.at[j], sem)
        pltpu.async_copy(bias_lv_hbm.at[idx_v.at[j]], blv_v.at[j], sem)
    pltpu.make_async_copy(lv_out.at[wid], lv_v, sem).wait()
    pltpu.make_async_copy(bmu_out.at[wid], bmu_v, sem).wait()
    pltpu.make_async_copy(blv_out.at[wid], blv_v, sem).wait()
    pltpu.sync_copy(lv_v, lv_out.at[wid])
    pltpu.sync_copy(bmu_v, bmu_out.at[wid])
    pltpu.sync_copy(blv_v, blv_out.at[wid])


_SC_PARAMS = pltpu.CompilerParams(
    needs_layout_passes=False, use_tc_tiling_on_sc=False)

_sc_gather_mu_call = functools.partial(
    pl.kernel,
    out_type=jax.ShapeDtypeStruct((NW, NCH * HALF, 128), jnp.float32),
    mesh=_SC_MESH,
    scratch_types=[
        pltpu.VMEM((NCH, 128), jnp.int32),
        pltpu.VMEM((HALF * NCH, 128), jnp.int32),
        pltpu.VMEM((NCH * HALF, 128), jnp.float32),
        pltpu.SemaphoreType.DMA,
    ],
    compiler_params=_SC_PARAMS,
)(_sc_gather_mu)

_sc_gather_lv_call = functools.partial(
    pl.kernel,
    out_type=(
        jax.ShapeDtypeStruct((NW, NCH * LATENT, 128), jnp.float32),
        jax.ShapeDtypeStruct((NW, NCH, 128), jnp.float32),
        jax.ShapeDtypeStruct((NW, NCH, 128), jnp.float32),
    ),
    mesh=_SC_MESH,
    scratch_types=[
        pltpu.VMEM((NCH, 128), jnp.int32),
        pltpu.VMEM((LATENT * NCH, 128), jnp.int32),
        pltpu.VMEM((NCH * LATENT, 128), jnp.float32),
        pltpu.VMEM((NCH, 128), jnp.float32),
        pltpu.VMEM((NCH, 128), jnp.float32),
        pltpu.SemaphoreType.DMA,
    ],
    compiler_params=_SC_PARAMS,
)(_sc_gather_lv)

_BLK = 4096
_WBLK = _BLK // BPW
_SUB = _BLK // 128


def _tc_decode_body(mua_ref, mub_ref, lv_ref, bmu_ref, blv_ref, eps_emb_ref,
                    eps_bias_ref, eps_dec_ref, wmu_ref, wlv_ref,
                    cbmu_ref, cblv_ref, out_ref):
    wmu = wmu_ref[...]
    wlv = wlv_ref[...]
    cbmu = cbmu_ref[...]
    cblv = cblv_ref[...]
    dnums = (((0,), (0,)), ((), ()))
    for s in range(_SUB):
        w, j = s // NCH, s % NCH
        cols = pl.ds(s * 128, 128)
        rows = pl.ds(s * 128, 128)
        bias = (bmu_ref[w, pl.ds(j, 1), :]
                + jnp.exp(0.5 * blv_ref[w, pl.ds(j, 1), :])
                * eps_bias_ref[:, cols])
        krows = pl.ds(j * LATENT, LATENT)
        hrows = pl.ds(j * HALF, HALF)
        mu = jnp.concatenate([mua_ref[w, hrows, :], mub_ref[w, hrows, :]],
                             axis=0)
        code = (mu
                + jnp.exp(0.5 * lv_ref[w, krows, :]) * eps_emb_ref[:, cols]
                + bias)
        mean = lax.dot_general(code, wmu, dnums,
                               preferred_element_type=jnp.float32)
        logvar = lax.dot_general(code, wlv, dnums,
                                 preferred_element_type=jnp.float32)
        out_ref[rows, :] = (mean + cbmu
                            + jnp.exp(0.5 * (logvar + cblv))
                            * eps_dec_ref[rows, :])


def _tc_decode(mu_a, mu_b, lv_t, bmu_t, blv_t, eps_emb_t, eps_bias_t,
               eps_dec, W_mu, W_lv, b_mu, b_lv):
    grid = (N // _BLK,)
    t4_blk = pl.BlockSpec((_WBLK, NCH * LATENT, 128), lambda i: (i, 0, 0))
    h4_blk = pl.BlockSpec((_WBLK, NCH * HALF, 128), lambda i: (i, 0, 0))
    g3_blk = pl.BlockSpec((_WBLK, NCH, 128), lambda i: (i, 0, 0))
    t_blk = pl.BlockSpec((LATENT, _BLK), lambda i: (0, i))
    r_blk = pl.BlockSpec((1, _BLK), lambda i: (0, i))
    w_blk = pl.BlockSpec((LATENT, NUM_METABOLITES), lambda i: (0, 0))
    b_blk = pl.BlockSpec((1, NUM_METABOLITES), lambda i: (0, 0))
    wide = pl.BlockSpec((_BLK, NUM_METABOLITES), lambda i: (i, 0))
    return pl.pallas_call(
        _tc_decode_body,
        grid=grid,
        in_specs=[h4_blk, h4_blk, t4_blk, g3_blk, g3_blk, t_blk, r_blk, wide,
                  w_blk, w_blk, b_blk, b_blk],
        out_specs=wide,
        out_shape=jax.ShapeDtypeStruct((N, NUM_METABOLITES), jnp.float32),
        compiler_params=pltpu.CompilerParams(
            dimension_semantics=("parallel",),
            fuse_transposed_lhs_in_matmul=True,
        ),
    )(mu_a, mu_b, lv_t, bmu_t, blv_t, eps_emb_t, eps_bias_t, eps_dec,
      W_mu, W_lv, b_mu, b_lv)


def kernel(x, emb_mu, emb_lv, bias_mu, bias_lv, W_mu, b_mu, W_lv, b_lv,
           eps_emb, eps_bias, eps_dec):
    x2 = x.reshape(-1, 128)
    emb_mu_t = emb_mu.T
    mu_a = _sc_gather_mu_call(x2, emb_mu_t[:HALF].reshape(-1))
    mu_b = _sc_gather_mu_call(x2, emb_mu_t[HALF:].reshape(-1))
    lv_t, bmu_g, blv_g = _sc_gather_lv_call(
        x2, emb_lv.T.reshape(-1),
        bias_mu.T.reshape(-1), bias_lv.T.reshape(-1),
    )
    return _tc_decode(
        mu_a, mu_b, lv_t, bmu_g, blv_g,
        eps_emb.T, eps_bias.reshape(1, N), eps_dec,
        W_mu, W_lv, b_mu.reshape(1, -1), b_lv.reshape(1, -1))


# speedup vs baseline: 1.1204x; 1.1204x over previous
import functools

import jax
import jax.numpy as jnp
from jax import lax
from jax.experimental import pallas as pl
from jax.experimental.pallas import tpu as pltpu
from jax.experimental.pallas import tpu_sc as plsc

N = 16384
V = 100000
LATENT = 16
NUM_METABOLITES = 128

NUM_CORES = 2
NUM_SUBCORES = 16
NW = NUM_CORES * NUM_SUBCORES
BPW = N // NW
NCH = BPW // 128

_SC_MESH = plsc.VectorSubcoreMesh(
    core_axis_name="c", subcore_axis_name="s",
    num_cores=NUM_CORES, num_subcores=NUM_SUBCORES,
)


def _build_offsets(idx_v, idxs_v):
    def off_body(r):
        j = r // LATENT
        k = r - j * LATENT
        for c in range(8):
            sl = pl.ds(c * 16, 16)
            idxs_v[r, sl] = idx_v[j, sl] + k * V
    plsc.parallel_loop(0, LATENT * NCH, 1, unroll=4)(off_body)


def _sc_gather_mu(x_hbm, tmu_hbm, mu_out, idx_v, idxs_v, mu_v, sem):
    wid = lax.axis_index("s") * NUM_CORES + lax.axis_index("c")
    pltpu.sync_copy(x_hbm.at[pl.ds(wid * NCH, NCH)], idx_v)
    _build_offsets(idx_v, idxs_v)

    def fire(r, carry):
        pltpu.async_copy(tmu_hbm.at[idxs_v.at[r]], mu_v.at[r], sem)
        return carry
    lax.fori_loop(0, LATENT * NCH, fire, 0, unroll=8)
    pltpu.make_async_copy(mu_out.at[wid], mu_v, sem).wait()
    pltpu.sync_copy(mu_v, mu_out.at[wid])


def _sc_gather_lv(x_hbm, tlv_hbm, bias_mu_hbm, bias_lv_hbm,
                  lv_out, bmu_out, blv_out,
                  idx_v, idxs_v, lv_v, bmu_v, blv_v, sem):
    wid = lax.axis_index("s") * NUM_CORES + lax.axis_index("c")
    pltpu.sync_copy(x_hbm.at[pl.ds(wid * NCH, NCH)], idx_v)
    _build_offsets(idx_v, idxs_v)

    def fire(r, carry):
        pltpu.async_copy(tlv_hbm.at[idxs_v.at[r]], lv_v.at[r], sem)
        return carry
    lax.fori_loop(0, LATENT * NCH, fire, 0, unroll=8)
    for j in range(NCH):
        pltpu.async_copy(bias_mu_hbm.at[idx_v.at[j]], bmu_v.at[j], sem)
        pltpu.async_copy(bias_lv_hbm.at[idx_v.at[j]], blv_v.at[j], sem)
    pltpu.make_async_copy(lv_out.at[wid], lv_v, sem).wait()
    pltpu.make_async_copy(bmu_out.at[wid], bmu_v, sem).wait()
    pltpu.make_async_copy(blv_out.at[wid], blv_v, sem).wait()
    pltpu.sync_copy(lv_v, lv_out.at[wid])
    pltpu.sync_copy(bmu_v, bmu_out.at[wid])
    pltpu.sync_copy(blv_v, blv_out.at[wid])


_SC_PARAMS = pltpu.CompilerParams(
    needs_layout_passes=False, use_tc_tiling_on_sc=False)

_sc_gather_mu_call = functools.partial(
    pl.kernel,
    out_type=jax.ShapeDtypeStruct((NW, NCH * LATENT, 128), jnp.float32),
    mesh=_SC_MESH,
    scratch_types=[
        pltpu.VMEM((NCH, 128), jnp.int32),
        pltpu.VMEM((LATENT * NCH, 128), jnp.int32),
        pltpu.VMEM((NCH * LATENT, 128), jnp.float32),
        pltpu.SemaphoreType.DMA,
    ],
    compiler_params=_SC_PARAMS,
)(_sc_gather_mu)

_sc_gather_lv_call = functools.partial(
    pl.kernel,
    out_type=(
        jax.ShapeDtypeStruct((NW, NCH * LATENT, 128), jnp.float32),
        jax.ShapeDtypeStruct((NW, NCH, 128), jnp.float32),
        jax.ShapeDtypeStruct((NW, NCH, 128), jnp.float32),
    ),
    mesh=_SC_MESH,
    scratch_types=[
        pltpu.VMEM((NCH, 128), jnp.int32),
        pltpu.VMEM((LATENT * NCH, 128), jnp.int32),
        pltpu.VMEM((NCH * LATENT, 128), jnp.float32),
        pltpu.VMEM((NCH, 128), jnp.float32),
        pltpu.VMEM((NCH, 128), jnp.float32),
        pltpu.SemaphoreType.DMA,
    ],
    compiler_params=_SC_PARAMS,
)(_sc_gather_lv)

_BLK = 8192
_WBLK = _BLK // BPW
_SUB = _BLK // 128


def _tc_decode_body(mu_ref, lv_ref, bmu_ref, blv_ref, eps_emb_ref,
                    eps_bias_ref, eps_dec_ref, wmu_ref, wlv_ref,
                    cbmu_ref, cblv_ref, out_ref):
    wmu = wmu_ref[...]
    wlv = wlv_ref[...]
    cbmu = cbmu_ref[...]
    cblv = cblv_ref[...]
    dnums = (((0,), (0,)), ((), ()))
    for s in range(_SUB):
        w, j = s // NCH, s % NCH
        cols = pl.ds(s * 128, 128)
        rows = pl.ds(s * 128, 128)
        bias = (bmu_ref[w, pl.ds(j, 1), :]
                + jnp.exp(0.5 * blv_ref[w, pl.ds(j, 1), :])
                * eps_bias_ref[:, cols])
        krows = pl.ds(j * LATENT, LATENT)
        code = (mu_ref[w, krows, :]
                + jnp.exp(0.5 * lv_ref[w, krows, :]) * eps_emb_ref[:, cols]
                + bias)
        mean = lax.dot_general(code, wmu, dnums,
                               preferred_element_type=jnp.float32)
        logvar = lax.dot_general(code, wlv, dnums,
                                 preferred_element_type=jnp.float32)
        out_ref[rows, :] = (mean + cbmu
                            + jnp.exp(0.5 * (logvar + cblv))
                            * eps_dec_ref[rows, :])


def _tc_decode(mu_t, lv_t, bmu_t, blv_t, eps_emb_t, eps_bias_t, eps_dec,
               W_mu, W_lv, b_mu, b_lv):
    grid = (N // _BLK,)
    t4_blk = pl.BlockSpec((_WBLK, NCH * LATENT, 128), lambda i: (i, 0, 0))
    g3_blk = pl.BlockSpec((_WBLK, NCH, 128), lambda i: (i, 0, 0))
    t_blk = pl.BlockSpec((LATENT, _BLK), lambda i: (0, i))
    r_blk = pl.BlockSpec((1, _BLK), lambda i: (0, i))
    w_blk = pl.BlockSpec((LATENT, NUM_METABOLITES), lambda i: (0, 0))
    b_blk = pl.BlockSpec((1, NUM_METABOLITES), lambda i: (0, 0))
    wide = pl.BlockSpec((_BLK, NUM_METABOLITES), lambda i: (i, 0))
    return pl.pallas_call(
        _tc_decode_body,
        grid=grid,
        in_specs=[t4_blk, t4_blk, g3_blk, g3_blk, t_blk, r_blk, wide,
                  w_blk, w_blk, b_blk, b_blk],
        out_specs=wide,
        out_shape=jax.ShapeDtypeStruct((N, NUM_METABOLITES), jnp.float32),
        compiler_params=pltpu.CompilerParams(
            dimension_semantics=("parallel",),
            fuse_transposed_lhs_in_matmul=True,
        ),
    )(mu_t, lv_t, bmu_t, blv_t, eps_emb_t, eps_bias_t, eps_dec,
      W_mu, W_lv, b_mu, b_lv)


def kernel(x, emb_mu, emb_lv, bias_mu, bias_lv, W_mu, b_mu, W_lv, b_lv,
           eps_emb, eps_bias, eps_dec):
    x2 = x.reshape(-1, 128)
    mu_t = _sc_gather_mu_call(x2, emb_mu.T.reshape(-1))
    lv_t, bmu_g, blv_g = _sc_gather_lv_call(
        x2, emb_lv.T.reshape(-1),
        bias_mu.T.reshape(-1), bias_lv.T.reshape(-1),
    )
    return _tc_decode(
        mu_t, lv_t, bmu_g, blv_g,
        eps_emb.T, eps_bias.reshape(1, N), eps_dec,
        W_mu, W_lv, b_mu.reshape(1, -1), b_lv.reshape(1, -1))
